# baseline (speedup 1.0000x reference)
import functools

import jax
import jax.numpy as jnp
from jax import lax
from jax.experimental import pallas as pl
from jax.experimental.pallas import tpu as pltpu
from jax.experimental.pallas import tpu_sc as plsc

_EMB_LEVELS = 1000000
_EMB_DIM = 16
_L = 16
_SEG = 128

_C1 = -1640531535
_C2 = 0x45D9F3B


def _hash_vec(xv):
    h = xv * jnp.int32(_C1)
    h = h ^ lax.shift_right_logical(h, 16)
    h = h * jnp.int32(_C2)
    h = h ^ lax.shift_right_logical(h, 16)
    q = lax.shift_right_logical(h, 1)
    r0 = h & jnp.int32(1)
    m = jnp.int32(_EMB_LEVELS)
    return lax.rem(lax.rem(q, m) * jnp.int32(2) + r0, m)


def _body(chunk, n_chunks, x_hbm, seeds_hbm, table_hbm, out_hbm,
          seeds_v, x_d, idx_d, rows_d, gsem, osem):
    info = plsc.get_sparse_core_info()
    nc = info.num_cores
    wid = lax.axis_index("s") * nc + lax.axis_index("c")
    per_w = chunk * n_chunks

    pltpu.sync_copy(seeds_hbm, seeds_v)
    s0 = seeds_v[0, :]
    s1 = seeds_v[1, :]
    m = jnp.int32(_EMB_LEVELS)

    def load_x(c):
        base = wid * per_w + c * chunk
        pltpu.sync_copy(x_hbm.at[pl.ds(base, chunk)], x_d.at[c & 1])

    def hash_chunk(c):
        slot = c & 1

        def grp(g, _):
            xv = x_d[slot, pl.ds(g * _L, _L)]
            h = _hash_vec(xv)
            idx_d[slot, 0, pl.ds(g * _L, _L)] = lax.rem(h + s0, m)
            idx_d[slot, 1, pl.ds(g * _L, _L)] = lax.rem(h + s1, m)
            return _

        lax.fori_loop(0, chunk // _L, grp, None, unroll=4)

    def fire_gathers(c):
        slot = c & 1
        return [
            pltpu.async_copy(table_hbm.at[idx_d.at[slot, s]],
                             rows_d.at[slot, s], gsem)
            for s in (0, 1)
        ]

    def fire_writes(c):
        base = wid * per_w + c * chunk
        slot = c & 1
        return [
            pltpu.async_copy(rows_d.at[slot, s],
                             out_hbm.at[pl.ds(base, chunk), s], osem)
            for s in (0, 1)
        ]

    def drain_writes(c):
        base = wid * per_w + c * chunk
        slot = c & 1
        for s in (0, 1):
            pltpu.make_async_copy(
                rows_d.at[slot, s],
                out_hbm.at[pl.ds(base, chunk), s], osem).wait()

    load_x(0)
    hash_chunk(0)

    def do_chunk(c, _):
        @pl.when(c >= 2)
        def _older():
            drain_writes(c - 2)

        gh = fire_gathers(c)
        cn = jnp.minimum(c + 1, n_chunks - 1)
        load_x(cn)
        hash_chunk(cn)
        for h in gh:
            h.wait()
        fire_writes(c)
        return _

    lax.fori_loop(0, n_chunks, do_chunk, None)
    drain_writes(n_chunks - 2)
    drain_writes(n_chunks - 1)


def kernel(x, fnum, table):
    batch, fields = x.shape
    fpad = 32
    n = batch * fpad
    x_flat = jnp.pad(x, ((0, 0), (0, fpad - fields))).reshape(n)
    seeds = jnp.broadcast_to(fnum.reshape(2, 1), (2, _L)).astype(jnp.int32)

    info = plsc.get_sparse_core_info()
    nw = info.num_cores * info.num_subcores
    per_w = n // nw
    assert per_w * nw == n
    chunk = 1024
    n_chunks = per_w // chunk
    assert n_chunks * chunk == per_w

    mesh = plsc.VectorSubcoreMesh(core_axis_name="c", subcore_axis_name="s")
    kfn = pl.kernel(
        functools.partial(_body, chunk, n_chunks),
        out_type=jax.ShapeDtypeStruct((batch * fpad, 8, _EMB_DIM),
                                      jnp.float32),
        mesh=mesh,
        compiler_params=pltpu.CompilerParams(use_tc_tiling_on_sc=False),
        scratch_types=[
            pltpu.VMEM((2, _L), jnp.int32),
            pltpu.VMEM((2, chunk), jnp.int32),
            pltpu.VMEM((2, 2, chunk), jnp.int32),
            pltpu.VMEM((2, 2, chunk, _EMB_DIM), jnp.float32),
            pltpu.SemaphoreType.DMA,
            pltpu.SemaphoreType.DMA,
        ],
    )
    out = kfn(x_flat, seeds, table)
    out = out.reshape(batch, fpad, 8 * _EMB_DIM)
    return out[:, :fields, :2 * _EMB_DIM]


# speedup vs baseline: 2.4095x; 2.4095x over previous
import functools

import jax
import jax.numpy as jnp
from jax import lax
from jax.experimental import pallas as pl
from jax.experimental.pallas import tpu as pltpu
from jax.experimental.pallas import tpu_sc as plsc

_EMB_LEVELS = 1000000
_EMB_DIM = 16
_L = 16
_SEG = 128

_C1 = -1640531535
_C2 = 0x45D9F3B


def _hash_vec(xv):
    h = xv * jnp.int32(_C1)
    h = h ^ lax.shift_right_logical(h, 16)
    h = h * jnp.int32(_C2)
    h = h ^ lax.shift_right_logical(h, 16)
    q = lax.shift_right_logical(h, 1)
    r0 = h & jnp.int32(1)
    m = jnp.int32(_EMB_LEVELS)
    return lax.rem(lax.rem(q, m) * jnp.int32(2) + r0, m)


def _body(chunk, n_chunks, x_hbm, seeds_hbm, table_hbm, out_hbm,
          seeds_v, x_d, idx_d, rows_d, gsem, osem):
    info = plsc.get_sparse_core_info()
    nc = info.num_cores
    wid = lax.axis_index("s") * nc + lax.axis_index("c")
    per_w = chunk * n_chunks

    pltpu.sync_copy(seeds_hbm, seeds_v)
    s0 = seeds_v[0, :]
    s1 = seeds_v[1, :]
    m = jnp.int32(_EMB_LEVELS)

    def load_x(c):
        base = wid * per_w + c * chunk
        pltpu.sync_copy(x_hbm.at[pl.ds(base, chunk)], x_d.at[c & 1])

    def hash_chunk(c):
        slot = c & 1

        def grp(g, _):
            xv = x_d[slot, pl.ds(g * _L, _L)]
            h = _hash_vec(xv)
            idx_d[slot, 0, pl.ds(g * _L, _L)] = lax.rem(h + s0, m)
            idx_d[slot, 1, pl.ds(g * _L, _L)] = lax.rem(h + s1, m)
            return _

        lax.fori_loop(0, chunk // _L, grp, None, unroll=4)

    def fire_gathers(c):
        slot = c & 1
        return [
            pltpu.async_copy(table_hbm.at[idx_d.at[slot, s]],
                             rows_d.at[slot, s], gsem)
            for s in (0, 1)
        ]

    def fire_writes(c):
        base = wid * per_w + c * chunk
        slot = c & 1
        return [
            pltpu.async_copy(rows_d.at[slot, s],
                             out_hbm.at[pl.ds(base, chunk), s], osem)
            for s in (0, 1)
        ]

    def drain_writes(c):
        base = wid * per_w + c * chunk
        slot = c & 1
        for s in (0, 1):
            pltpu.make_async_copy(
                rows_d.at[slot, s],
                out_hbm.at[pl.ds(base, chunk), s], osem).wait()

    load_x(0)
    hash_chunk(0)

    def do_chunk(c, _):
        @pl.when(c >= 2)
        def _older():
            drain_writes(c - 2)

        gh = fire_gathers(c)
        cn = jnp.minimum(c + 1, n_chunks - 1)
        load_x(cn)
        hash_chunk(cn)
        for h in gh:
            h.wait()
        fire_writes(c)
        return _

    lax.fori_loop(0, n_chunks, do_chunk, None)
    drain_writes(n_chunks - 2)
    drain_writes(n_chunks - 1)


def kernel(x, fnum, table):
    batch, fields = x.shape
    fpad = 32
    n = batch * fpad
    x_flat = jnp.concatenate([x, x[:, :fpad - fields]], axis=1).reshape(n)
    seeds = jnp.broadcast_to(fnum.reshape(2, 1), (2, _L)).astype(jnp.int32)

    info = plsc.get_sparse_core_info()
    nw = info.num_cores * info.num_subcores
    per_w = n // nw
    assert per_w * nw == n
    chunk = 1024
    n_chunks = per_w // chunk
    assert n_chunks * chunk == per_w

    mesh = plsc.VectorSubcoreMesh(core_axis_name="c", subcore_axis_name="s")
    kfn = pl.kernel(
        functools.partial(_body, chunk, n_chunks),
        out_type=jax.ShapeDtypeStruct((batch * fpad, 8, _EMB_DIM),
                                      jnp.float32),
        mesh=mesh,
        compiler_params=pltpu.CompilerParams(use_tc_tiling_on_sc=False),
        scratch_types=[
            pltpu.VMEM((2, _L), jnp.int32),
            pltpu.VMEM((2, chunk), jnp.int32),
            pltpu.VMEM((2, 2, chunk), jnp.int32),
            pltpu.VMEM((2, 2, chunk, _EMB_DIM), jnp.float32),
            pltpu.SemaphoreType.DMA,
            pltpu.SemaphoreType.DMA,
        ],
    )
    out = kfn(x_flat, seeds, table)
    out = out.reshape(batch, fpad, 8 * _EMB_DIM)
    return out[:, :fields, :2 * _EMB_DIM]
